# baseline (speedup 1.0000x reference)
import functools

import jax
import jax.numpy as jnp
from jax import lax
from jax.experimental import pallas as pl
from jax.experimental.pallas import tpu as pltpu
from jax.experimental.pallas import tpu_sc as plsc

B, N, MC = 8, 1024, 16
F = 256
CONV = 512
NUM_LABELS = 128
_CHUNK = 128


def _sc_embedding_gather(emb, node2d):
    info = plsc.get_sparse_core_info()
    nc, ns = info.num_cores, info.num_subcores
    nw = nc * ns
    total_chunks = (B * N) // _CHUNK
    chunks_per_w = total_chunks // nw
    mesh = plsc.VectorSubcoreMesh(core_axis_name="c", subcore_axis_name="s")

    @functools.partial(
        pl.kernel,
        mesh=mesh,
        out_type=jax.ShapeDtypeStruct((B * N, F), jnp.float32),
        scratch_types=[
            pltpu.VMEM((chunks_per_w, _CHUNK), jnp.int32),
            pltpu.VMEM((chunks_per_w, _CHUNK, F), jnp.float32),
            pltpu.SemaphoreType.DMA,
        ],
    )
    def gather_kernel(emb_hbm, idx_hbm, out_hbm, idx_v, rows_v, sem):
        wid = lax.axis_index("s") * nc + lax.axis_index("c")
        first = wid * chunks_per_w
        pltpu.sync_copy(idx_hbm.at[pl.ds(first, chunks_per_w)], idx_v)
        copies = [
            pltpu.async_copy(emb_hbm.at[idx_v.at[j]], rows_v.at[j], sem)
            for j in range(chunks_per_w)
        ]
        for c in copies:
            c.wait()
        for j in range(chunks_per_w):
            pltpu.sync_copy(
                rows_v.at[j], out_hbm.at[pl.ds((first + j) * _CHUNK, _CHUNK)]
            )

    return gather_kernel(emb, node2d)


def _tc_body(ch_ref, nodes_ref, w_all_ref, bc_ref, whl_ref, bhl_ref, out_ref,
             a_r_ref, a_m_ref, cr_ref):
    ch = ch_ref[0]
    nodes = nodes_ref[0]

    maskf = jnp.where(ch != 0, 1.0, 0.0)
    nsib = jnp.sum(maskf, axis=0, keepdims=True)
    single = nsib == 1.0
    denom = jnp.where(single, 1.0, nsib - 1.0)
    islot = lax.broadcasted_iota(jnp.int32, (MC, N), 0)
    cidx = islot.astype(jnp.float32)
    slot0 = islot == 0
    r_reg = cidx / denom
    r_one = jnp.where(slot0, 0.5, 0.0)
    r_coef = jnp.where(single, r_one, r_reg)
    cr_ref[...] = r_coef

    iota16 = lax.broadcasted_iota(jnp.int16, (N, N), 0)

    def slot_planes(c):
        ch_c = ch_ref[0, pl.ds(c, 1), :].astype(jnp.int16)
        eq = iota16 == ch_c
        r_row = cr_ref[pl.ds(c, 1), :].astype(jnp.bfloat16)
        dr = jnp.where(eq, r_row, jnp.bfloat16(0.0))
        dm = eq.astype(jnp.bfloat16)
        return dr, dm

    dr0, dm0 = slot_planes(0)
    a_r_ref[...] = dr0
    a_m_ref[...] = dm0

    def body(c, carry):
        dr, dm = slot_planes(c)
        a_r_ref[...] += dr
        a_m_ref[...] += dm
        return carry

    lax.fori_loop(1, MC, body, 0)

    row0 = lax.broadcasted_iota(jnp.int32, (N, F), 0) == 0
    nodes0 = jnp.where(row0, 0.0, nodes).astype(jnp.bfloat16)

    dn = (((0,), (0,)), ((), ()))
    comb_r = lax.dot_general(a_r_ref[...], nodes0, dn,
                             preferred_element_type=jnp.float32)
    s_sum = lax.dot_general(a_m_ref[...], nodes0, dn,
                            preferred_element_type=jnp.float32)
    comb_l = s_sum - comb_r

    x = jnp.concatenate([nodes, comb_r, comb_l], axis=1)
    y = jnp.dot(x, w_all_ref[...], preferred_element_type=jnp.float32)
    conv = jnp.tanh(y + bc_ref[...])
    pooled = jnp.max(conv, axis=0, keepdims=True)
    out = lax.dot_general(pooled, whl_ref[...], (((1,), (1,)), ((), ())),
                          preferred_element_type=jnp.float32)
    out_ref[0] = out + bhl_ref[...]


def _tc_conv(ch_t, nodes3, w_all, b_conv2, w_hl, b_hl2, interpret=False):
    return pl.pallas_call(
        _tc_body,
        grid=(B,),
        in_specs=[
            pl.BlockSpec((1, MC, N), lambda b: (b, 0, 0)),
            pl.BlockSpec((1, N, F), lambda b: (b, 0, 0)),
            pl.BlockSpec((3 * F, CONV), lambda b: (0, 0)),
            pl.BlockSpec((1, CONV), lambda b: (0, 0)),
            pl.BlockSpec((NUM_LABELS, CONV), lambda b: (0, 0)),
            pl.BlockSpec((1, NUM_LABELS), lambda b: (0, 0)),
        ],
        out_specs=pl.BlockSpec((1, 1, NUM_LABELS), lambda b: (b, 0, 0)),
        out_shape=jax.ShapeDtypeStruct((B, 1, NUM_LABELS), jnp.float32),
        scratch_shapes=[
            pltpu.VMEM((N, N), jnp.bfloat16),
            pltpu.VMEM((N, N), jnp.bfloat16),
            pltpu.VMEM((MC, N), jnp.float32),
        ],
        compiler_params=pltpu.CompilerParams(
            dimension_semantics=("arbitrary",)),
        interpret=interpret,
    )(ch_t, nodes3, w_all, b_conv2, w_hl, b_hl2)


def kernel(node, children, emb, w_t, w_l, w_r, b_conv, w_hl, b_hl):
    node2d = node.reshape((B * N) // _CHUNK, _CHUNK).astype(jnp.int32)
    nodes = _sc_embedding_gather(emb, node2d)
    nodes3 = nodes.reshape(B, N, F)
    ch_t = jnp.swapaxes(children, 1, 2).astype(jnp.int32)
    w_all = (jnp.concatenate([w_t, w_r, w_l], axis=0)
             .reshape(F, 3, CONV).swapaxes(0, 1).reshape(3 * F, CONV))
    out3 = _tc_conv(ch_t, nodes3, w_all, b_conv.reshape(1, CONV), w_hl,
                    b_hl.reshape(1, NUM_LABELS))
    return out3.reshape(B, NUM_LABELS)


# speedup vs baseline: 1.2511x; 1.2511x over previous
import functools

import jax
import jax.numpy as jnp
from jax import lax
from jax.experimental import pallas as pl
from jax.experimental.pallas import tpu as pltpu
from jax.experimental.pallas import tpu_sc as plsc

B, N, MC = 8, 1024, 16
F = 256
CONV = 512
NUM_LABELS = 128
_CHUNK = 128


def _sc_embedding_gather(emb, node2d):
    info = plsc.get_sparse_core_info()
    nc, ns = info.num_cores, info.num_subcores
    nw = nc * ns
    total_chunks = (B * N) // _CHUNK
    chunks_per_w = total_chunks // nw
    mesh = plsc.VectorSubcoreMesh(core_axis_name="c", subcore_axis_name="s")

    @functools.partial(
        pl.kernel,
        mesh=mesh,
        out_type=jax.ShapeDtypeStruct((B * N, F), jnp.float32),
        scratch_types=[
            pltpu.VMEM((chunks_per_w, _CHUNK), jnp.int32),
            pltpu.VMEM((chunks_per_w, _CHUNK, F), jnp.float32),
            pltpu.SemaphoreType.DMA,
        ],
    )
    def gather_kernel(emb_hbm, idx_hbm, out_hbm, idx_v, rows_v, sem):
        wid = lax.axis_index("s") * nc + lax.axis_index("c")
        first = wid * chunks_per_w
        pltpu.sync_copy(idx_hbm.at[pl.ds(first, chunks_per_w)], idx_v)
        copies = [
            pltpu.async_copy(emb_hbm.at[idx_v.at[j]], rows_v.at[j], sem)
            for j in range(chunks_per_w)
        ]
        for c in copies:
            c.wait()
        for j in range(chunks_per_w):
            pltpu.sync_copy(
                rows_v.at[j], out_hbm.at[pl.ds((first + j) * _CHUNK, _CHUNK)]
            )

    return gather_kernel(emb, node2d)


def _tc_body(ch_ref, nodes_ref, w_all_ref, bc_ref, whl_ref, bhl_ref, out_ref,
             a_r_ref, a_m_ref, cr_ref):
    ch = ch_ref[0]
    nodes = nodes_ref[0]

    maskf = jnp.where(ch != 0, 1.0, 0.0)
    nsib = jnp.sum(maskf, axis=0, keepdims=True)
    single = nsib == 1.0
    denom = jnp.where(single, 1.0, nsib - 1.0)
    islot = lax.broadcasted_iota(jnp.int32, (MC, N), 0)
    cidx = islot.astype(jnp.float32)
    slot0 = islot == 0
    r_reg = cidx / denom
    r_one = jnp.where(slot0, 0.5, 0.0)
    r_coef = jnp.where(single, r_one, r_reg)
    cr_ref[...] = r_coef

    iota_m = lax.broadcasted_iota(jnp.int32, (N, N), 0)

    def slot_planes(c):
        ch_c = ch_ref[0, pl.ds(c, 1), :]
        eq = iota_m == ch_c
        r_row = cr_ref[pl.ds(c, 1), :]
        dr = jnp.where(eq, r_row, 0.0)
        dm = jnp.where(eq, 1.0, 0.0)
        return dr, dm

    dr0, dm0 = slot_planes(0)
    a_r_ref[...] = dr0
    a_m_ref[...] = dm0

    def body(c, carry):
        dr, dm = slot_planes(c)
        a_r_ref[...] += dr
        a_m_ref[...] += dm
        return carry

    lax.fori_loop(1, MC, body, 0)

    row0 = lax.broadcasted_iota(jnp.int32, (N, F), 0) == 0
    nodes0 = jnp.where(row0, 0.0, nodes)

    dn = (((0,), (0,)), ((), ()))
    comb_r = lax.dot_general(a_r_ref[...], nodes0, dn,
                             preferred_element_type=jnp.float32)
    s_sum = lax.dot_general(a_m_ref[...], nodes0, dn,
                            preferred_element_type=jnp.float32)
    comb_l = s_sum - comb_r

    x = jnp.concatenate([nodes, comb_r, comb_l], axis=1)
    y = jnp.dot(x, w_all_ref[...], preferred_element_type=jnp.float32)
    conv = jnp.tanh(y + bc_ref[...])
    pooled = jnp.max(conv, axis=0, keepdims=True)
    out = lax.dot_general(pooled, whl_ref[...], (((1,), (1,)), ((), ())),
                          preferred_element_type=jnp.float32)
    out_ref[0] = out + bhl_ref[...]


def _tc_conv(ch_t, nodes3, w_all, b_conv2, w_hl, b_hl2, interpret=False):
    return pl.pallas_call(
        _tc_body,
        grid=(B,),
        in_specs=[
            pl.BlockSpec((1, MC, N), lambda b: (b, 0, 0)),
            pl.BlockSpec((1, N, F), lambda b: (b, 0, 0)),
            pl.BlockSpec((3 * F, CONV), lambda b: (0, 0)),
            pl.BlockSpec((1, CONV), lambda b: (0, 0)),
            pl.BlockSpec((NUM_LABELS, CONV), lambda b: (0, 0)),
            pl.BlockSpec((1, NUM_LABELS), lambda b: (0, 0)),
        ],
        out_specs=pl.BlockSpec((1, 1, NUM_LABELS), lambda b: (b, 0, 0)),
        out_shape=jax.ShapeDtypeStruct((B, 1, NUM_LABELS), jnp.float32),
        scratch_shapes=[
            pltpu.VMEM((N, N), jnp.float32),
            pltpu.VMEM((N, N), jnp.float32),
            pltpu.VMEM((MC, N), jnp.float32),
        ],
        compiler_params=pltpu.CompilerParams(
            dimension_semantics=("arbitrary",)),
        interpret=interpret,
    )(ch_t, nodes3, w_all, b_conv2, w_hl, b_hl2)


def kernel(node, children, emb, w_t, w_l, w_r, b_conv, w_hl, b_hl):
    node2d = node.reshape((B * N) // _CHUNK, _CHUNK).astype(jnp.int32)
    nodes = _sc_embedding_gather(emb, node2d)
    nodes3 = nodes.reshape(B, N, F)
    ch_t = jnp.swapaxes(children, 1, 2).astype(jnp.int32)
    w_all = (jnp.concatenate([w_t, w_r, w_l], axis=0)
             .reshape(F, 3, CONV).swapaxes(0, 1).reshape(3 * F, CONV))
    out3 = _tc_conv(ch_t, nodes3, w_all, b_conv.reshape(1, CONV), w_hl,
                    b_hl.reshape(1, NUM_LABELS))
    return out3.reshape(B, NUM_LABELS)
